# Initial kernel scaffold; baseline (speedup 1.0000x reference)
import jax
import jax.numpy as jnp
from jax.experimental import pallas as pl


def kernel(edge_index, x, node_indicator, W1, b1, W2, b2, Wc, bc):
    raise NotImplementedError("write your pallas kernel here")


# speedup vs baseline: 9.5665x; 9.5665x over previous
import functools

import jax
import jax.numpy as jnp
from jax import lax
from jax.experimental import pallas as pl
from jax.experimental.pallas import tpu as pltpu, tpu_sc as plsc

N_NODES = 10000
N_EDGES = 320000
D_FEAT = 128
HIDDEN = 32
DENSE = 512
NUM_CLASSES = 10
NUM_GRAPHS = 16

NC = 2
NS = 16
NW = NC * NS

CHUNK = 128
EPW = 10240
E_PAD = EPW * NW
NCHUNK = EPW // CHUNK
RPT = 632
NPAD = RPT * NS


def _mm_body(x_ref, w_ref, b_ref, o_ref):
    o_ref[...] = (
        jnp.dot(x_ref[...], w_ref[...], preferred_element_type=jnp.float32)
        + b_ref[...]
    )


def _matmul(x_p, W1, b1):
    return pl.pallas_call(
        _mm_body,
        out_shape=jax.ShapeDtypeStruct((NPAD, HIDDEN), jnp.float32),
    )(x_p, W1, b1.reshape(1, HIDDEN))


def _sc_body(src_hbm, dst_hbm, h_hbm, zeros_hbm, out_hbm,
             sidx_v, didx_v, rows_v, slab_v, agg_sh, sem):
    c = lax.axis_index("c")
    s = lax.axis_index("s")
    wid = c * NS + s

    pltpu.sync_copy(zeros_hbm.at[pl.ds(s * RPT, RPT)], slab_v)
    pltpu.sync_copy(slab_v, agg_sh.at[pl.ds(s * RPT, RPT)])
    pltpu.sync_copy(src_hbm.at[wid], sidx_v)
    pltpu.sync_copy(dst_hbm.at[wid], didx_v)
    plsc.subcore_barrier()

    def chunk(j, carry):
        pltpu.async_copy(h_hbm.at[sidx_v.at[j]], rows_v, sem).wait()
        pltpu.sync_copy(rows_v, agg_sh.at[didx_v.at[j]], add=True)
        return carry

    lax.fori_loop(0, NCHUNK, chunk, 0)
    plsc.subcore_barrier()
    pltpu.sync_copy(agg_sh.at[pl.ds(s * RPT, RPT)], slab_v)
    pltpu.sync_copy(slab_v, out_hbm.at[pl.ds(c * NPAD + s * RPT, RPT)])


@functools.partial(
    pl.kernel,
    out_type=jax.ShapeDtypeStruct((NC * NPAD, HIDDEN), jnp.float32),
    mesh=plsc.VectorSubcoreMesh(core_axis_name="c", subcore_axis_name="s"),
    compiler_params=pltpu.CompilerParams(use_tc_tiling_on_sc=False),
    scratch_types=[
        pltpu.VMEM((NCHUNK, CHUNK), jnp.int32),
        pltpu.VMEM((NCHUNK, CHUNK), jnp.int32),
        pltpu.VMEM((CHUNK, HIDDEN), jnp.float32),
        pltpu.VMEM((RPT, HIDDEN), jnp.float32),
        pltpu.VMEM_SHARED((NPAD, HIDDEN), jnp.float32),
        pltpu.SemaphoreType.DMA,
    ],
)
def _sc_aggregate(src_hbm, dst_hbm, h_hbm, zeros_hbm, out_hbm,
                  sidx_v, didx_v, rows_v, slab_v, agg_sh, sem):
    _sc_body(src_hbm, dst_hbm, h_hbm, zeros_hbm, out_hbm,
             sidx_v, didx_v, rows_v, slab_v, agg_sh, sem)


def _post_body(aggp_ref, ind_ref, w2_ref, b2_ref, wc_ref, bc_ref, o_ref):
    a = jnp.maximum(aggp_ref[0] + aggp_ref[1], 0.0)
    gids = lax.broadcasted_iota(jnp.int32, (NUM_GRAPHS, NPAD), 0)
    m = (ind_ref[...] == gids).astype(jnp.float32)
    pooled = jnp.dot(m, a, preferred_element_type=jnp.float32)
    z = jnp.maximum(
        jnp.dot(pooled, w2_ref[...], preferred_element_type=jnp.float32)
        + b2_ref[...], 0.0)
    logits = (jnp.dot(z, wc_ref[...], preferred_element_type=jnp.float32)
              + bc_ref[...])
    mx = jnp.max(logits, axis=-1, keepdims=True)
    e = jnp.exp(logits - mx)
    o_ref[...] = e / jnp.sum(e, axis=-1, keepdims=True)


def _post(agg_pair, ind_p, W2, b2, Wc, bc):
    return pl.pallas_call(
        _post_body,
        out_shape=jax.ShapeDtypeStruct((NUM_GRAPHS, NUM_CLASSES), jnp.float32),
    )(agg_pair, ind_p, W2, b2.reshape(1, DENSE), Wc, bc.reshape(1, NUM_CLASSES))


def kernel(edge_index, x, node_indicator, W1, b1, W2, b2, Wc, bc):
    src = edge_index[0].astype(jnp.int32)
    dst = edge_index[1].astype(jnp.int32)
    pad = E_PAD - src.shape[0]
    src_p = jnp.concatenate(
        [src, jnp.full((pad,), N_NODES, jnp.int32)]).reshape(NW, NCHUNK, CHUNK)
    dst_p = jnp.concatenate(
        [dst, jnp.full((pad,), N_NODES, jnp.int32)]).reshape(NW, NCHUNK, CHUNK)

    x_p = jnp.pad(x, ((0, NPAD - N_NODES), (0, 0)))
    h = _matmul(x_p, W1, b1)

    zeros = jnp.zeros((NPAD, HIDDEN), jnp.float32)
    agg_pair = _sc_aggregate(src_p, dst_p, h, zeros).reshape(NC, NPAD, HIDDEN)

    ind_p = jnp.pad(node_indicator.astype(jnp.int32),
                    (0, NPAD - N_NODES),
                    constant_values=NUM_GRAPHS).reshape(1, NPAD)
    return _post(agg_pair, ind_p, W2, b2, Wc, bc)
